# baseline (speedup 1.0000x reference)
import functools

import jax
import jax.numpy as jnp
import numpy as np
from jax.experimental import pallas as pl
from jax.experimental.pallas import tpu as pltpu
from jax.experimental.shard_map import shard_map
from jax.sharding import Mesh, PartitionSpec as P

E = 16
TOP_K = 2
H = 1024
FF = 1024
ALPHA = 1.702
LIMIT = 7.0
NEG = -1e30


def _moe_kernel(off_ref, x_ref, rw_ref, rb_ref, gw_ref, gb_ref, uw_ref,
                ub_ref, dw_ref, db_ref, out_ref, scores_ref, *, e_loc):
    e = pl.program_id(0)

    @pl.when(e == 0)
    def _router():
        x = x_ref[...]
        logits = jax.lax.dot_general(
            x, rw_ref[...], (((1,), (1,)), ((), ())),
            preferred_element_type=jnp.float32) + rb_ref[...]
        iota = jax.lax.broadcasted_iota(jnp.int32, logits.shape, 1)
        m1 = jnp.max(logits, axis=1, keepdims=True)
        idx1 = jnp.min(jnp.where(logits == m1, iota, E), axis=1, keepdims=True)
        mask1 = iota == idx1
        rest = jnp.where(mask1, NEG, logits)
        m2 = jnp.max(rest, axis=1, keepdims=True)
        idx2 = jnp.min(jnp.where(rest == m2, iota, E), axis=1, keepdims=True)
        mask2 = iota == idx2
        p1 = 1.0 / (1.0 + jnp.exp(m2 - m1))
        p2 = 1.0 - p1
        scores_ref[...] = jnp.where(mask1, p1, 0.0) + jnp.where(mask2, p2, 0.0)

    xb = x_ref[...].astype(jnp.bfloat16)
    gate = jax.lax.dot_general(
        xb, gw_ref[0].astype(jnp.bfloat16), (((1,), (1,)), ((), ())),
        preferred_element_type=jnp.float32) + gb_ref[0]
    up = jax.lax.dot_general(
        xb, uw_ref[0].astype(jnp.bfloat16), (((1,), (1,)), ((), ())),
        preferred_element_type=jnp.float32) + ub_ref[0]
    gate = jnp.minimum(gate, LIMIT)
    up = jnp.clip(up, -LIMIT, LIMIT)
    glu = gate * jax.nn.sigmoid(gate * ALPHA)
    act = (up + 1.0) * glu
    y = jax.lax.dot_general(
        act.astype(jnp.bfloat16), dw_ref[0].astype(jnp.bfloat16),
        (((1,), (1,)), ((), ())),
        preferred_element_type=jnp.float32) + db_ref[0]
    s = scores_ref[...]
    cols = jax.lax.broadcasted_iota(jnp.int32, s.shape, 1)
    w = jnp.sum(jnp.where(cols == e + off_ref[0, 0], s, 0.0),
                axis=1, keepdims=True)
    contrib = w * y

    @pl.when(e == 0)
    def _init():
        out_ref[...] = contrib

    @pl.when(e != 0)
    def _acc():
        out_ref[...] += contrib


def _run_local(off2, x, rw, rb2, gw, gb3, uw, ub3, dw, db3, *, e_loc, ttok):
    return pl.pallas_call(
        functools.partial(_moe_kernel, e_loc=e_loc),
        grid=(e_loc,),
        in_specs=[
            pl.BlockSpec((1, 1), lambda e: (0, 0)),
            pl.BlockSpec((ttok, H), lambda e: (0, 0)),
            pl.BlockSpec((E, H), lambda e: (0, 0)),
            pl.BlockSpec((1, E), lambda e: (0, 0)),
            pl.BlockSpec((1, FF, H), lambda e: (e, 0, 0)),
            pl.BlockSpec((1, 1, FF), lambda e: (e, 0, 0)),
            pl.BlockSpec((1, FF, H), lambda e: (e, 0, 0)),
            pl.BlockSpec((1, 1, FF), lambda e: (e, 0, 0)),
            pl.BlockSpec((1, H, FF), lambda e: (e, 0, 0)),
            pl.BlockSpec((1, 1, H), lambda e: (e, 0, 0)),
        ],
        out_specs=[
            pl.BlockSpec((ttok, H), lambda e: (0, 0)),
            pl.BlockSpec((ttok, E), lambda e: (0, 0)),
        ],
        out_shape=[
            jax.ShapeDtypeStruct((ttok, H), jnp.float32),
            jax.ShapeDtypeStruct((ttok, E), jnp.float32),
        ],
        compiler_params=pltpu.CompilerParams(
            dimension_semantics=("arbitrary",),
            vmem_limit_bytes=100 * 1024 * 1024,
        ),
    )(off2, x, rw, rb2, gw, gb3, uw, ub3, dw, db3)


@functools.partial(jax.jit, static_argnums=())
def kernel(hidden_states, router_w, router_b, gate_w, gate_b, up_w, up_b,
           down_w, down_b):
    Bn, Tn, Hn = hidden_states.shape
    x = hidden_states.reshape(-1, Hn)
    ttok = x.shape[0]
    rb2 = router_b.reshape(1, E)
    gb3 = gate_b.reshape(E, 1, FF)
    ub3 = up_b.reshape(E, 1, FF)
    db3 = down_b.reshape(E, 1, H)

    devs = jax.devices()
    ndev = len(devs)
    nd = 1
    while nd * 2 <= min(ndev, E) and E % (nd * 2) == 0:
        nd *= 2
    e_loc = E // nd

    if nd == 1:
        off2 = jnp.zeros((1, 1), jnp.int32)
        out, scores = _run_local(off2, x, router_w, rb2, gate_w, gb3, up_w,
                                 ub3, down_w, db3, e_loc=E, ttok=ttok)
        return out.reshape(Bn, Tn, Hn), scores

    mesh = Mesh(np.array(devs[:nd]), ("x",))

    def shard_fn(x, rw, rb2, gw, gb3, uw, ub3, dw, db3):
        off = jax.lax.axis_index("x").astype(jnp.int32) * e_loc
        off2 = jnp.full((1, 1), off, jnp.int32)
        out_p, scores = _run_local(off2, x, rw, rb2, gw, gb3, uw, ub3, dw,
                                   db3, e_loc=e_loc, ttok=ttok)
        out = jax.lax.psum(out_p, "x")
        return out, scores

    out, scores = shard_map(
        shard_fn, mesh=mesh,
        in_specs=(P(), P(), P(), P("x"), P("x"), P("x"), P("x"), P("x"),
                  P("x")),
        out_specs=(P(), P()),
        check_rep=False,
    )(x, router_w, rb2, gate_w, gb3, up_w, ub3, down_w, db3)

    return out.reshape(Bn, Tn, Hn), scores


# speedup vs baseline: 5.3681x; 5.3681x over previous
import functools

import jax
import jax.numpy as jnp
from jax.experimental import pallas as pl
from jax.experimental.pallas import tpu as pltpu

E = 16
TOP_K = 2
H = 1024
FF = 1024
ALPHA = 1.702
LIMIT = 7.0
NEG = -1e30
EPB = 2


def _moe_kernel(x_ref, rw_ref, rb_ref, gw_ref, gb_ref, uw_ref, ub_ref,
                dw_ref, db_ref, out_ref, scores_ref):
    e = pl.program_id(0)

    @pl.when(e == 0)
    def _router():
        x = x_ref[...]
        logits = jax.lax.dot_general(
            x, rw_ref[...], (((1,), (1,)), ((), ())),
            preferred_element_type=jnp.float32) + rb_ref[...]
        iota = jax.lax.broadcasted_iota(jnp.int32, logits.shape, 1)
        m1 = jnp.max(logits, axis=1, keepdims=True)
        idx1 = jnp.min(jnp.where(logits == m1, iota, E), axis=1, keepdims=True)
        mask1 = iota == idx1
        rest = jnp.where(mask1, NEG, logits)
        m2 = jnp.max(rest, axis=1, keepdims=True)
        idx2 = jnp.min(jnp.where(rest == m2, iota, E), axis=1, keepdims=True)
        mask2 = iota == idx2
        p1 = 1.0 / (1.0 + jnp.exp(m2 - m1))
        p2 = 1.0 - p1
        scores_ref[...] = jnp.where(mask1, p1, 0.0) + jnp.where(mask2, p2, 0.0)

    xb = x_ref[...].astype(jnp.bfloat16)
    s = scores_ref[...]
    cols = jax.lax.broadcasted_iota(jnp.int32, s.shape, 1)
    contrib = None
    for i in range(EPB):
        gate = jax.lax.dot_general(
            xb, gw_ref[i].astype(jnp.bfloat16), (((1,), (1,)), ((), ())),
            preferred_element_type=jnp.float32) + gb_ref[i]
        up = jax.lax.dot_general(
            xb, uw_ref[i].astype(jnp.bfloat16), (((1,), (1,)), ((), ())),
            preferred_element_type=jnp.float32) + ub_ref[i]
        gate = jnp.minimum(gate, LIMIT)
        up = jnp.clip(up, -LIMIT, LIMIT)
        glu = gate * jax.nn.sigmoid(gate * ALPHA)
        act = (up + 1.0) * glu
        y = jax.lax.dot_general(
            act.astype(jnp.bfloat16), dw_ref[i].astype(jnp.bfloat16),
            (((1,), (1,)), ((), ())),
            preferred_element_type=jnp.float32) + db_ref[i]
        w = jnp.sum(jnp.where(cols == EPB * e + i, s, 0.0),
                    axis=1, keepdims=True)
        contrib = w * y if contrib is None else contrib + w * y

    @pl.when(e == 0)
    def _init():
        out_ref[...] = contrib

    @pl.when(e != 0)
    def _acc():
        out_ref[...] += contrib


@functools.partial(jax.jit, static_argnums=())
def kernel(hidden_states, router_w, router_b, gate_w, gate_b, up_w, up_b,
           down_w, down_b):
    Bn, Tn, Hn = hidden_states.shape
    x = hidden_states.reshape(-1, Hn)
    Ttok = x.shape[0]
    rb2 = router_b.reshape(1, E)
    gb3 = gate_b.reshape(E, 1, FF)
    ub3 = up_b.reshape(E, 1, FF)
    db3 = down_b.reshape(E, 1, H)

    out, scores = pl.pallas_call(
        _moe_kernel,
        grid=(E // EPB,),
        in_specs=[
            pl.BlockSpec((Ttok, H), lambda e: (0, 0)),
            pl.BlockSpec((E, H), lambda e: (0, 0)),
            pl.BlockSpec((1, E), lambda e: (0, 0)),
            pl.BlockSpec((EPB, FF, H), lambda e: (e, 0, 0)),
            pl.BlockSpec((EPB, 1, FF), lambda e: (e, 0, 0)),
            pl.BlockSpec((EPB, FF, H), lambda e: (e, 0, 0)),
            pl.BlockSpec((EPB, 1, FF), lambda e: (e, 0, 0)),
            pl.BlockSpec((EPB, H, FF), lambda e: (e, 0, 0)),
            pl.BlockSpec((EPB, 1, H), lambda e: (e, 0, 0)),
        ],
        out_specs=[
            pl.BlockSpec((Ttok, H), lambda e: (0, 0)),
            pl.BlockSpec((Ttok, E), lambda e: (0, 0)),
        ],
        out_shape=[
            jax.ShapeDtypeStruct((Ttok, H), jnp.float32),
            jax.ShapeDtypeStruct((Ttok, E), jnp.float32),
        ],
        compiler_params=pltpu.CompilerParams(
            dimension_semantics=("arbitrary",),
            vmem_limit_bytes=100 * 1024 * 1024,
        ),
    )(x, router_w, rb2, gate_w, gb3, up_w, ub3, down_w, db3)

    return out.reshape(Bn, Tn, Hn), scores


# speedup vs baseline: 5.6210x; 1.0471x over previous
import functools

import jax
import jax.numpy as jnp
from jax.experimental import pallas as pl
from jax.experimental.pallas import tpu as pltpu

E = 16
TOP_K = 2
H = 1024
FF = 1024
ALPHA = 1.702
LIMIT = 7.0
NEG = -1e30


def _moe_kernel(x_ref, rw_ref, rb_ref, gwa_ref, gwb_ref, uwa_ref, uwb_ref,
                dwa_ref, dwb_ref, gb_ref, ub_ref, db_ref, out_ref,
                scores_ref):
    e = pl.program_id(0)

    @pl.when(e == 0)
    def _router():
        x = x_ref[...]
        logits = jax.lax.dot_general(
            x, rw_ref[...], (((1,), (1,)), ((), ())),
            preferred_element_type=jnp.float32) + rb_ref[...]
        iota = jax.lax.broadcasted_iota(jnp.int32, logits.shape, 1)
        m1 = jnp.max(logits, axis=1, keepdims=True)
        idx1 = jnp.min(jnp.where(logits == m1, iota, E), axis=1, keepdims=True)
        mask1 = iota == idx1
        rest = jnp.where(mask1, NEG, logits)
        m2 = jnp.max(rest, axis=1, keepdims=True)
        idx2 = jnp.min(jnp.where(rest == m2, iota, E), axis=1, keepdims=True)
        mask2 = iota == idx2
        p1 = 1.0 / (1.0 + jnp.exp(m2 - m1))
        p2 = 1.0 - p1
        scores_ref[...] = jnp.where(mask1, p1, 0.0) + jnp.where(mask2, p2, 0.0)

    xb = x_ref[...].astype(jnp.bfloat16)
    s = scores_ref[...]
    cols = jax.lax.broadcasted_iota(jnp.int32, s.shape, 1)
    contrib = None
    for i, (gw_ref, uw_ref, dw_ref) in enumerate(
            ((gwa_ref, uwa_ref, dwa_ref), (gwb_ref, uwb_ref, dwb_ref))):
        gate = jax.lax.dot_general(
            xb, gw_ref[0].astype(jnp.bfloat16), (((1,), (1,)), ((), ())),
            preferred_element_type=jnp.float32) + gb_ref[i]
        up = jax.lax.dot_general(
            xb, uw_ref[0].astype(jnp.bfloat16), (((1,), (1,)), ((), ())),
            preferred_element_type=jnp.float32) + ub_ref[i]
        gate = jnp.minimum(gate, LIMIT)
        up = jnp.clip(up, -LIMIT, LIMIT)
        glu = gate * jax.nn.sigmoid(gate * ALPHA)
        act = (up + 1.0) * glu
        y = jax.lax.dot_general(
            act.astype(jnp.bfloat16), dw_ref[0].astype(jnp.bfloat16),
            (((1,), (1,)), ((), ())),
            preferred_element_type=jnp.float32) + db_ref[i]
        w = jnp.sum(jnp.where(cols == 2 * e + i, s, 0.0),
                    axis=1, keepdims=True)
        contrib = w * y if contrib is None else contrib + w * y

    @pl.when(e == 0)
    def _init():
        out_ref[...] = contrib

    @pl.when(e != 0)
    def _acc():
        out_ref[...] += contrib


@functools.partial(jax.jit, static_argnums=())
def kernel(hidden_states, router_w, router_b, gate_w, gate_b, up_w, up_b,
           down_w, down_b):
    Bn, Tn, Hn = hidden_states.shape
    x = hidden_states.reshape(-1, Hn)
    Ttok = x.shape[0]
    rb2 = router_b.reshape(1, E)
    gb3 = gate_b.reshape(E, 1, FF)
    ub3 = up_b.reshape(E, 1, FF)
    db3 = down_b.reshape(E, 1, H)

    wspec_a = pl.BlockSpec((1, FF, H), lambda e: (2 * e, 0, 0))
    wspec_b = pl.BlockSpec((1, FF, H), lambda e: (2 * e + 1, 0, 0))

    out, scores = pl.pallas_call(
        _moe_kernel,
        grid=(E // 2,),
        in_specs=[
            pl.BlockSpec((Ttok, H), lambda e: (0, 0)),
            pl.BlockSpec((E, H), lambda e: (0, 0)),
            pl.BlockSpec((1, E), lambda e: (0, 0)),
            wspec_a,
            wspec_b,
            wspec_a,
            wspec_b,
            wspec_a,
            wspec_b,
            pl.BlockSpec((2, 1, FF), lambda e: (e, 0, 0)),
            pl.BlockSpec((2, 1, FF), lambda e: (e, 0, 0)),
            pl.BlockSpec((2, 1, H), lambda e: (e, 0, 0)),
        ],
        out_specs=[
            pl.BlockSpec((Ttok, H), lambda e: (0, 0)),
            pl.BlockSpec((Ttok, E), lambda e: (0, 0)),
        ],
        out_shape=[
            jax.ShapeDtypeStruct((Ttok, H), jnp.float32),
            jax.ShapeDtypeStruct((Ttok, E), jnp.float32),
        ],
        compiler_params=pltpu.CompilerParams(
            dimension_semantics=("arbitrary",),
            vmem_limit_bytes=100 * 1024 * 1024,
        ),
    )(x, router_w, rb2, gate_w, gate_w, up_w, up_w, down_w, down_w,
      gb3, ub3, db3)

    return out.reshape(Bn, Tn, Hn), scores
